# baseline (speedup 1.0000x reference)
import functools

import jax
import jax.numpy as jnp
from jax import lax
from jax.experimental import pallas as pl
from jax.experimental.pallas import tpu as pltpu
from jax.experimental.pallas import tpu_sc as plsc

B = 16384
D = 16
H = 8
NW = 32
BW = B // NW
EW = 16
RND = 2 * EW
NG = BW // RND

P_B1 = 384
P_W2 = 392
P_B2 = 400


@functools.partial(
    pl.kernel,
    mesh=plsc.VectorSubcoreMesh(core_axis_name="c", subcore_axis_name="s"),
    out_type=jax.ShapeDtypeStruct((B,), jnp.float32),
    scratch_types=[
        pltpu.VMEM((BW,), jnp.int32),
        pltpu.VMEM((BW,), jnp.int32),
        pltpu.VMEM((D, EW * 128), jnp.float32),
        pltpu.VMEM((D, EW * 128), jnp.float32),
        pltpu.VMEM((D, EW * 128), jnp.float32),
        pltpu.VMEM((D, 8 * 128), jnp.float32),
        pltpu.VMEM((D, RND), jnp.float32),
        pltpu.VMEM((D, RND), jnp.float32),
        pltpu.VMEM((4, 128), jnp.float32),
        pltpu.VMEM((BW,), jnp.float32),
        pltpu.SemaphoreType.DMA,
        pltpu.SemaphoreType.DMA,
        pltpu.SemaphoreType.DMA,
        pltpu.SemaphoreType.DMA,
    ],
    compiler_params=pltpu.CompilerParams(needs_layout_passes=False),
)
def _hhgr_sc(uin_hbm, iin_hbm, utabt_hbm, itabt_hbm, params_hbm, out_hbm,
             uidx_v, iidx_v, blk0_v, blk1_v, blk2_v, blk3_v,
             ustg_v, istg_v, pv, out_v, sem0, sem1, sem2, sem3):
    wid = lax.axis_index("s") * 2 + lax.axis_index("c")
    base = wid * BW

    pltpu.sync_copy(uin_hbm.at[pl.ds(base, BW)], uidx_v)
    pltpu.sync_copy(iin_hbm.at[pl.ds(base, BW)], iidx_v)
    pltpu.sync_copy(params_hbm, pv)

    iota = lax.iota(jnp.int32, 16)
    bufs = [blk0_v, blk1_v, blk2_v, blk3_v]
    sems = [sem0, sem1, sem2, sem3]

    def fire(tab_hbm, idxvecs, half, w, lo=0, n=EW):
        for e in range(lo, lo + n):
            blkbase = pl.multiple_of(
                idxvecs[half][e] & ~jnp.int32(127), 128)
            pltpu.async_copy(
                tab_hbm.at[:, pl.ds(blkbase, 128)],
                bufs[w].at[:, pl.ds((e - lo) * 128, 128)], sems[w])

    def drain(w, n=EW):
        pltpu.make_async_copy(utabt_hbm.at[:, pl.ds(0, n * 128)],
                              bufs[w], sems[w]).wait()

    def ext(lanevecs, half, w, stg, lo=0, n=EW):
        for e in range(lo, lo + n):
            col = jnp.full((16,), (e - lo) * 128, jnp.int32) \
                + lanevecs[half][e]
            vals = plsc.load_gather(bufs[w], [iota, col])
            plsc.store_scatter(
                stg, [iota, jnp.full((16,), half * EW + e, jnp.int32)], vals)

    def wb(w):
        return plsc.load_gather(
            pv, [jnp.full((16,), w // 128, jnp.int32),
                 jnp.full((16,), w % 128, jnp.int32)])

    def idxs(g):
        uv = [uidx_v[pl.ds(g * RND, 16)], uidx_v[pl.ds(g * RND + 16, 16)]]
        iv = [iidx_v[pl.ds(g * RND, 16)], iidx_v[pl.ds(g * RND + 16, 16)]]
        return uv, iv

    uv0, iv0 = idxs(0)
    fire(utabt_hbm, uv0, 0, 0)
    fire(utabt_hbm, uv0, 1, 1)
    fire(itabt_hbm, iv0, 0, 2)
    fire(itabt_hbm, iv0, 1, 3, 0, 8)

    def rnd(g, carry):
        uv, iv = idxs(g)
        ulane = [v & 127 for v in uv]
        ilane = [v & 127 for v in iv]
        uvn, ivn = idxs(jnp.minimum(g + 1, NG - 1))

        drain(0)
        ext(ulane, 0, 0, ustg_v)
        fire(utabt_hbm, uvn, 0, 0)
        drain(1)
        ext(ulane, 1, 1, ustg_v)
        fire(utabt_hbm, uvn, 1, 1)
        drain(2)
        ext(ilane, 0, 2, istg_v)
        fire(itabt_hbm, ivn, 0, 2)
        drain(3, 8)
        ext(ilane, 1, 3, istg_v, 0, 8)
        fire(itabt_hbm, iv, 1, 3, 8, 8)
        drain(3, 8)
        ext(ilane, 1, 3, istg_v, 8, 8)
        fire(itabt_hbm, ivn, 1, 3, 0, 8)

        for grp in range(2):
            h = [wb(P_B1 + j) for j in range(H)]
            for d in range(D):
                u_d = ustg_v[d, pl.ds(grp * 16, 16)]
                i_d = istg_v[d, pl.ds(grp * 16, 16)]
                e_d = u_d * i_d
                for j in range(H):
                    h[j] = (h[j]
                            + e_d * wb(d * H + j)
                            + u_d * wb((D + d) * H + j)
                            + i_d * wb((2 * D + d) * H + j))
            logit = wb(P_B2)
            for j in range(H):
                logit = logit + jnp.maximum(h[j], 0.0) * wb(P_W2 + j)
            out_v[pl.ds(g * RND + grp * 16, 16)] = (
                1.0 / (1.0 + jnp.exp(-logit)))
        return carry

    lax.fori_loop(0, NG, rnd, 0)

    for w in range(3):
        drain(w)
    drain(3, 8)
    pltpu.sync_copy(out_v, out_hbm.at[pl.ds(base, BW)])


def kernel(user_inputs, item_inputs, user_table, item_table, W1, b1, W2, b2):
    flat = jnp.concatenate([
        W1.reshape(-1),
        b1.reshape(-1),
        W2.reshape(-1),
        b2.reshape(-1),
        jnp.zeros((111,), jnp.float32),
    ])
    params = flat.reshape(4, 128)
    y = _hhgr_sc(user_inputs.astype(jnp.int32), item_inputs.astype(jnp.int32),
                 user_table.T, item_table.T, params)
    return y.reshape(B, 1)


# speedup vs baseline: 1.3509x; 1.3509x over previous
import functools

import jax
import jax.numpy as jnp
from jax import lax
from jax.experimental import pallas as pl
from jax.experimental.pallas import tpu as pltpu
from jax.experimental.pallas import tpu_sc as plsc

B = 16384
D = 16
H = 8
NW = 32
BW = B // NW
NG = BW // 16

P_B1 = 384
P_W2 = 392
P_B2 = 400


@functools.partial(
    pl.kernel,
    mesh=plsc.VectorSubcoreMesh(core_axis_name="c", subcore_axis_name="s"),
    out_type=jax.ShapeDtypeStruct((B,), jnp.float32),
    scratch_types=[
        pltpu.VMEM((BW,), jnp.int32),
        pltpu.VMEM((BW,), jnp.int32),
        pltpu.VMEM((D, 8 * 128), jnp.float32),
        pltpu.VMEM((D, 8 * 128), jnp.float32),
        pltpu.VMEM((D, 8 * 128), jnp.float32),
        pltpu.VMEM((D, 8 * 128), jnp.float32),
        pltpu.VMEM((D, 16), jnp.float32),
        pltpu.VMEM((D, 16), jnp.float32),
        pltpu.VMEM((4, 128), jnp.float32),
        pltpu.VMEM((BW,), jnp.float32),
        pltpu.SemaphoreType.DMA,
        pltpu.SemaphoreType.DMA,
        pltpu.SemaphoreType.DMA,
        pltpu.SemaphoreType.DMA,
    ],
    compiler_params=pltpu.CompilerParams(needs_layout_passes=False),
)
def _hhgr_sc(uin_hbm, iin_hbm, utabt_hbm, itabt_hbm, params_hbm, out_hbm,
             uidx_v, iidx_v, blk0_v, blk1_v, blk2_v, blk3_v,
             ustg_v, istg_v, pv, out_v, sem0, sem1, sem2, sem3):
    wid = lax.axis_index("s") * 2 + lax.axis_index("c")
    base = wid * BW

    pltpu.sync_copy(uin_hbm.at[pl.ds(base, BW)], uidx_v)
    pltpu.sync_copy(iin_hbm.at[pl.ds(base, BW)], iidx_v)
    pltpu.sync_copy(params_hbm, pv)

    iota = lax.iota(jnp.int32, 16)
    bufs = [blk0_v, blk1_v, blk2_v, blk3_v]
    sems = [sem0, sem1, sem2, sem3]

    def fire(tab_hbm, idxvec, half, w):
        for e in range(8):
            blkbase = pl.multiple_of(
                idxvec[half * 8 + e] & ~jnp.int32(127), 128)
            pltpu.async_copy(
                tab_hbm.at[:, pl.ds(blkbase, 128)],
                bufs[w].at[:, pl.ds(e * 128, 128)], sems[w])

    def drain(w):
        pltpu.make_async_copy(utabt_hbm.at[:, pl.ds(0, 8 * 128)],
                              bufs[w], sems[w]).wait()

    def ext(lanevec, half, w, stg):
        for e in range(8):
            col = jnp.full((16,), e * 128, jnp.int32) + lanevec[half * 8 + e]
            vals = plsc.load_gather(bufs[w], [iota, col])
            plsc.store_scatter(
                stg, [iota, jnp.full((16,), half * 8 + e, jnp.int32)], vals)

    def wb(w):
        return plsc.load_gather(
            pv, [jnp.full((16,), w // 128, jnp.int32),
                 jnp.full((16,), w % 128, jnp.int32)])

    def fire_round(uv, iv):
        fire(utabt_hbm, uv, 0, 0)
        fire(utabt_hbm, uv, 1, 1)
        fire(itabt_hbm, iv, 0, 2)
        fire(itabt_hbm, iv, 1, 3)

    fire_round(uidx_v[pl.ds(0, 16)], iidx_v[pl.ds(0, 16)])

    def rnd(g, carry):
        uv = uidx_v[pl.ds(g * 16, 16)]
        iv = iidx_v[pl.ds(g * 16, 16)]
        ulane = uv & 127
        ilane = iv & 127
        gn = jnp.minimum(g + 1, NG - 1)
        uvn = uidx_v[pl.ds(gn * 16, 16)]
        ivn = iidx_v[pl.ds(gn * 16, 16)]

        drain(0)
        ext(ulane, 0, 0, ustg_v)
        fire(utabt_hbm, uvn, 0, 0)
        drain(1)
        ext(ulane, 1, 1, ustg_v)
        fire(utabt_hbm, uvn, 1, 1)
        drain(2)
        ext(ilane, 0, 2, istg_v)
        fire(itabt_hbm, ivn, 0, 2)
        drain(3)
        ext(ilane, 1, 3, istg_v)
        fire(itabt_hbm, ivn, 1, 3)

        h = [wb(P_B1 + j) for j in range(H)]
        for d in range(D):
            u_d = ustg_v[d]
            i_d = istg_v[d]
            e_d = u_d * i_d
            for j in range(H):
                h[j] = (h[j]
                        + e_d * wb(d * H + j)
                        + u_d * wb((D + d) * H + j)
                        + i_d * wb((2 * D + d) * H + j))
        logit = wb(P_B2)
        for j in range(H):
            logit = logit + jnp.maximum(h[j], 0.0) * wb(P_W2 + j)
        out_v[pl.ds(g * 16, 16)] = 1.0 / (1.0 + jnp.exp(-logit))
        return carry

    lax.fori_loop(0, NG, rnd, 0)

    for w in range(4):
        drain(w)
    pltpu.sync_copy(out_v, out_hbm.at[pl.ds(base, BW)])


def kernel(user_inputs, item_inputs, user_table, item_table, W1, b1, W2, b2):
    flat = jnp.concatenate([
        W1.reshape(-1),
        b1.reshape(-1),
        W2.reshape(-1),
        b2.reshape(-1),
        jnp.zeros((111,), jnp.float32),
    ])
    params = flat.reshape(4, 128)
    y = _hhgr_sc(user_inputs.astype(jnp.int32), item_inputs.astype(jnp.int32),
                 user_table.T, item_table.T, params)
    return y.reshape(B, 1)


# speedup vs baseline: 1.3522x; 1.0010x over previous
import functools

import jax
import jax.numpy as jnp
from jax import lax
from jax.experimental import pallas as pl
from jax.experimental.pallas import tpu as pltpu
from jax.experimental.pallas import tpu_sc as plsc

B = 16384
D = 16
H = 8
NW = 32
BW = B // NW
NG = BW // 16

P_B1 = 384
P_W2 = 392
P_B2 = 400


@functools.partial(
    pl.kernel,
    mesh=plsc.VectorSubcoreMesh(core_axis_name="c", subcore_axis_name="s"),
    out_type=jax.ShapeDtypeStruct((B,), jnp.float32),
    scratch_types=[
        pltpu.VMEM((BW,), jnp.int32),
        pltpu.VMEM((BW,), jnp.int32),
        pltpu.VMEM((D, 8 * 128), jnp.float32),
        pltpu.VMEM((D, 8 * 128), jnp.float32),
        pltpu.VMEM((D, 8 * 128), jnp.float32),
        pltpu.VMEM((D, 8 * 128), jnp.float32),
        pltpu.VMEM((D, 16), jnp.float32),
        pltpu.VMEM((D, 16), jnp.float32),
        pltpu.VMEM((4, 128), jnp.float32),
        pltpu.VMEM((BW,), jnp.float32),
        pltpu.SemaphoreType.DMA,
        pltpu.SemaphoreType.DMA,
        pltpu.SemaphoreType.DMA,
        pltpu.SemaphoreType.DMA,
    ],
    compiler_params=pltpu.CompilerParams(needs_layout_passes=False),
)
def _hhgr_sc(uin_hbm, iin_hbm, utabt_hbm, itabt_hbm, params_hbm, out_hbm,
             uidx_v, iidx_v, blk0_v, blk1_v, blk2_v, blk3_v,
             ustg_v, istg_v, pv, out_v, sem0, sem1, sem2, sem3):
    wid = lax.axis_index("s") * 2 + lax.axis_index("c")
    base = wid * BW

    pltpu.sync_copy(uin_hbm.at[pl.ds(base, BW)], uidx_v)
    pltpu.sync_copy(iin_hbm.at[pl.ds(base, BW)], iidx_v)
    pltpu.sync_copy(params_hbm, pv)

    iota = lax.iota(jnp.int32, 16)
    bufs = [blk0_v, blk1_v, blk2_v, blk3_v]
    sems = [sem0, sem1, sem2, sem3]

    def fire(tab_hbm, idxvec, half, w):
        for e in range(8):
            blkbase = pl.multiple_of(
                idxvec[half * 8 + e] & ~jnp.int32(127), 128)
            pltpu.async_copy(
                tab_hbm.at[pl.ds(0, 8), pl.ds(blkbase, 128)],
                bufs[w].at[pl.ds(0, 8), pl.ds(e * 128, 128)], sems[w])
            pltpu.async_copy(
                tab_hbm.at[pl.ds(8, 8), pl.ds(blkbase, 128)],
                bufs[w].at[pl.ds(8, 8), pl.ds(e * 128, 128)], sems[w])

    def drain(w):
        pltpu.make_async_copy(utabt_hbm.at[:, pl.ds(0, 8 * 128)],
                              bufs[w], sems[w]).wait()

    def ext(lanevec, half, w, stg):
        for e in range(8):
            col = jnp.full((16,), e * 128, jnp.int32) + lanevec[half * 8 + e]
            vals = plsc.load_gather(bufs[w], [iota, col])
            plsc.store_scatter(
                stg, [iota, jnp.full((16,), half * 8 + e, jnp.int32)], vals)

    def wb(w):
        return plsc.load_gather(
            pv, [jnp.full((16,), w // 128, jnp.int32),
                 jnp.full((16,), w % 128, jnp.int32)])

    def fire_round(uv, iv):
        fire(utabt_hbm, uv, 0, 0)
        fire(utabt_hbm, uv, 1, 1)
        fire(itabt_hbm, iv, 0, 2)
        fire(itabt_hbm, iv, 1, 3)

    fire_round(uidx_v[pl.ds(0, 16)], iidx_v[pl.ds(0, 16)])

    def rnd(g, carry):
        uv = uidx_v[pl.ds(g * 16, 16)]
        iv = iidx_v[pl.ds(g * 16, 16)]
        ulane = uv & 127
        ilane = iv & 127
        gn = jnp.minimum(g + 1, NG - 1)
        uvn = uidx_v[pl.ds(gn * 16, 16)]
        ivn = iidx_v[pl.ds(gn * 16, 16)]

        drain(0)
        ext(ulane, 0, 0, ustg_v)
        fire(utabt_hbm, uvn, 0, 0)
        drain(1)
        ext(ulane, 1, 1, ustg_v)
        fire(utabt_hbm, uvn, 1, 1)
        drain(2)
        ext(ilane, 0, 2, istg_v)
        fire(itabt_hbm, ivn, 0, 2)
        drain(3)
        ext(ilane, 1, 3, istg_v)
        fire(itabt_hbm, ivn, 1, 3)

        h = [wb(P_B1 + j) for j in range(H)]
        for d in range(D):
            u_d = ustg_v[d]
            i_d = istg_v[d]
            e_d = u_d * i_d
            for j in range(H):
                h[j] = (h[j]
                        + e_d * wb(d * H + j)
                        + u_d * wb((D + d) * H + j)
                        + i_d * wb((2 * D + d) * H + j))
        logit = wb(P_B2)
        for j in range(H):
            logit = logit + jnp.maximum(h[j], 0.0) * wb(P_W2 + j)
        out_v[pl.ds(g * 16, 16)] = 1.0 / (1.0 + jnp.exp(-logit))
        return carry

    lax.fori_loop(0, NG, rnd, 0)

    for w in range(4):
        drain(w)
    pltpu.sync_copy(out_v, out_hbm.at[pl.ds(base, BW)])


def kernel(user_inputs, item_inputs, user_table, item_table, W1, b1, W2, b2):
    flat = jnp.concatenate([
        W1.reshape(-1),
        b1.reshape(-1),
        W2.reshape(-1),
        b2.reshape(-1),
        jnp.zeros((111,), jnp.float32),
    ])
    params = flat.reshape(4, 128)
    y = _hhgr_sc(user_inputs.astype(jnp.int32), item_inputs.astype(jnp.int32),
                 user_table.T, item_table.T, params)
    return y.reshape(B, 1)
